# baseline (speedup 1.0000x reference)
import jax
import jax.numpy as jnp
from jax import lax
from jax.experimental import pallas as pl
from jax.experimental.pallas import tpu as pltpu
from jax.experimental.pallas import tpu_sc as plsc

_B = 16384
_S = 50
_D = 32
_NC = 2
_NS = 16
_NW = _NC * _NS
_BPW = _B // _NW
_WAVE = 32
_NWAVE = _BPW // _WAVE


def _sc_title_cat_body(cat_hbm, title_t_hbm, cat_tbl, title_tbl,
                       cat_out, pooled_out,
                       cidx_v, tidx_v, cat_rows_v, pooled_v,
                       sem_t, sem_c):
    wid = lax.axis_index("s") * _NC + lax.axis_index("c")
    base = wid * _BPW

    pltpu.sync_copy(cat_hbm.at[pl.ds(base, _BPW)], cidx_v)
    pltpu.sync_copy(title_t_hbm.at[:, pl.ds(base, _BPW)], tidx_v)

    cat_cp = pltpu.async_copy(cat_tbl.at[cidx_v], cat_rows_v, sem_c)

    zeros = jnp.zeros((16,), jnp.float32)

    def zero_body(r, carry):
        pooled_v[r, 0:16] = zeros
        pooled_v[r, 16:32] = zeros
        return carry

    lax.fori_loop(0, _BPW, zero_body, 0, unroll=8)

    def fire_body(s, carry):
        pltpu.async_copy(title_tbl.at[tidx_v.at[s]], pooled_v, sem_t, add=True)
        return carry

    lax.fori_loop(0, _S, fire_body, 0)

    def drain_body(s, carry):
        pltpu.make_async_copy(
            title_tbl.at[tidx_v.at[0]], pooled_v, sem_t).wait()
        return carry

    lax.fori_loop(0, _S, drain_body, 0)

    cat_cp.wait()

    w1 = pltpu.async_copy(cat_rows_v, cat_out.at[wid], sem_c)
    w2 = pltpu.async_copy(pooled_v, pooled_out.at[wid], sem_t)
    w1.wait()
    w2.wait()


_sc_title_cat = pl.kernel(
    _sc_title_cat_body,
    out_type=(
        jax.ShapeDtypeStruct((_NW, _BPW, _D), jnp.float32),
        jax.ShapeDtypeStruct((_NW, _BPW, _D), jnp.float32),
    ),
    mesh=plsc.VectorSubcoreMesh(core_axis_name="c", subcore_axis_name="s"),
    compiler_params=pltpu.CompilerParams(use_tc_tiling_on_sc=False),
    scratch_types=[
        pltpu.VMEM((_BPW,), jnp.int32),
        pltpu.VMEM((_S, _BPW), jnp.int32),
        pltpu.VMEM((_BPW, _D), jnp.float32),
        pltpu.VMEM((_BPW, _D), jnp.float32),
        pltpu.SemaphoreType.DMA,
        pltpu.SemaphoreType.DMA,
    ],
)


_ISTEP = 4
_NSTEP = _BPW // _ISTEP


def _sc_item_body(item_id_hbm, item_tbl_t, item_out,
                  iidx_v, buf_a, buf_b, rows_v, sem_w, sem_o):
    wid = lax.axis_index("s") * _NC + lax.axis_index("c")
    base = wid * _BPW
    iota = lax.broadcasted_iota(jnp.int32, (16,), 0)

    pltpu.sync_copy(item_id_hbm.at[pl.ds(base, _BPW)],
                    iidx_v.at[pl.ds(0, _BPW)])

    def fire(k, p, buf):
        vec = iidx_v[pl.ds(k * 2 * _ISTEP, 16)]
        for j in range(_ISTEP):
            cb = pl.multiple_of((vec[p * _ISTEP + j] // 128) * 128, 128)
            for g in range(_D // 8):
                pltpu.async_copy(
                    item_tbl_t.at[pl.ds(g * 8, 8), pl.ds(cb, 128)],
                    buf.at[pl.ds(g * 8, 8), pl.ds(j * 128, 128)], sem_w)

    def finish(k, p, buf):
        pltpu.make_async_copy(item_tbl_t.at[:, pl.ds(0, _ISTEP * 128)],
                              buf, sem_w).wait()
        vec = iidx_v[pl.ds(k * 2 * _ISTEP, 16)]
        for j in range(_ISTEP):
            row = vec[p * _ISTEP + j]
            lane = row - (row // 128) * 128
            lane_v = jnp.full((16,), lane, jnp.int32)
            slot = (2 * k + p) * _ISTEP + j
            lane_j = lane_v + j * 128
            rows_v[pl.ds(slot * _D, 16)] = plsc.load_gather(
                buf, [iota, lane_j])
            rows_v[pl.ds(slot * _D + 16, 16)] = plsc.load_gather(
                buf, [iota + 16, lane_j])

    fire(0, 0, buf_a)

    def body(k, carry):
        fire(k, 1, buf_b)
        finish(k, 0, buf_a)

        @pl.when(k < _NSTEP // 2 - 1)
        def _():
            fire(k + 1, 0, buf_a)

        finish(k, 1, buf_b)
        nw = 2 * _ISTEP * _D
        pltpu.async_copy(
            rows_v.at[pl.ds(k * nw, nw)],
            item_out.at[pl.ds(base * _D + k * nw, nw)], sem_o)
        return carry

    lax.fori_loop(0, _NSTEP // 2, body, 0)

    pltpu.make_async_copy(rows_v, item_out.at[pl.ds(0, _BPW * _D)],
                          sem_o).wait()


_sc_item = pl.kernel(
    _sc_item_body,
    out_type=jax.ShapeDtypeStruct((_B * _D,), jnp.float32),
    mesh=plsc.VectorSubcoreMesh(core_axis_name="c", subcore_axis_name="s"),
    compiler_params=pltpu.CompilerParams(use_tc_tiling_on_sc=True,
                                         needs_layout_passes=False),
    scratch_types=[
        pltpu.VMEM((_BPW + 16, ), jnp.int32),
        pltpu.VMEM((_D, _ISTEP * 128), jnp.float32),
        pltpu.VMEM((_D, _ISTEP * 128), jnp.float32),
        pltpu.VMEM((_BPW * _D,), jnp.float32),
        pltpu.SemaphoreType.DMA,
        pltpu.SemaphoreType.DMA,
    ],
)


def _dense_body(item_ref, cat_ref, pooled_ref, wi_ref, wc_ref, wp_ref,
                b_ref, out_ref):
    y = (jnp.dot(item_ref[...], wi_ref[...],
                 preferred_element_type=jnp.float32)
         + jnp.dot(cat_ref[...], wc_ref[...],
                   preferred_element_type=jnp.float32)
         + jnp.dot(pooled_ref[...], wp_ref[...],
                   preferred_element_type=jnp.float32))
    out_ref[...] = jnp.maximum(y + b_ref[...], 0.0)


_PK = 128 // _D


def kernel(item_id, category, title, item_table, category_table, title_table,
           W, b):
    cat_emb, pooled = _sc_title_cat(
        category.astype(jnp.int32),
        title.astype(jnp.int32).T,
        category_table, title_table)
    item_emb = _sc_item(item_id.astype(jnp.int32), item_table.T)

    b4 = _B // _PK
    item4 = item_emb.reshape(b4, _PK * _D)
    cat4 = cat_emb.reshape(b4, _PK * _D)
    pooled4 = pooled.reshape(b4, _PK * _D)

    eye = jnp.eye(_PK, dtype=jnp.float32)
    wi4 = jnp.kron(eye, W[0:_D, :])
    wc4 = jnp.kron(eye, W[_D:2 * _D, :])
    wp4 = jnp.kron(eye, W[2 * _D:, :] * (1.0 / _S))
    b4v = jnp.tile(b, _PK).reshape(1, _PK * 64)

    bb4 = 1024
    dense = pl.pallas_call(
        _dense_body,
        grid=(b4 // bb4,),
        in_specs=[
            pl.BlockSpec((bb4, _PK * _D), lambda i: (i, 0)),
            pl.BlockSpec((bb4, _PK * _D), lambda i: (i, 0)),
            pl.BlockSpec((bb4, _PK * _D), lambda i: (i, 0)),
            pl.BlockSpec((_PK * _D, _PK * 64), lambda i: (0, 0)),
            pl.BlockSpec((_PK * _D, _PK * 64), lambda i: (0, 0)),
            pl.BlockSpec((_PK * _D, _PK * 64), lambda i: (0, 0)),
            pl.BlockSpec((1, _PK * 64), lambda i: (0, 0)),
        ],
        out_specs=pl.BlockSpec((bb4, _PK * 64), lambda i: (i, 0)),
        out_shape=jax.ShapeDtypeStruct((b4, _PK * 64), jnp.float32),
    )
    out4 = dense(item4, cat4, pooled4, wi4, wc4, wp4, b4v)
    return out4.reshape(_B, 64)


# speedup vs baseline: 1.0028x; 1.0028x over previous
import jax
import jax.numpy as jnp
from jax import lax
from jax.experimental import pallas as pl
from jax.experimental.pallas import tpu as pltpu
from jax.experimental.pallas import tpu_sc as plsc

_B = 16384
_S = 50
_D = 32
_NC = 2
_NS = 16
_NW = _NC * _NS
_BPW = _B // _NW


def _sc_title_cat_body(cat_hbm, title_t_hbm, cat_tbl, title_tbl,
                       cat_out, pooled_out,
                       cidx_v, tidx_v, cat_rows_v, pooled_v,
                       sem_t, sem_c):
    wid = lax.axis_index("s") * _NC + lax.axis_index("c")
    base = wid * _BPW

    pltpu.sync_copy(cat_hbm.at[pl.ds(base, _BPW)], cidx_v)
    pltpu.sync_copy(title_t_hbm.at[:, pl.ds(base, _BPW)], tidx_v)

    cat_cp = pltpu.async_copy(cat_tbl.at[cidx_v], cat_rows_v, sem_c)

    zeros = jnp.zeros((16,), jnp.float32)

    def zero_body(r, carry):
        pooled_v[r, 0:16] = zeros
        pooled_v[r, 16:32] = zeros
        return carry

    lax.fori_loop(0, _BPW, zero_body, 0, unroll=8)

    def fire_body(s, carry):
        pltpu.async_copy(title_tbl.at[tidx_v.at[s]], pooled_v, sem_t, add=True)
        return carry

    lax.fori_loop(0, _S, fire_body, 0)

    def drain_body(s, carry):
        pltpu.make_async_copy(
            title_tbl.at[tidx_v.at[0]], pooled_v, sem_t).wait()
        return carry

    lax.fori_loop(0, _S, drain_body, 0)

    cat_cp.wait()

    w1 = pltpu.async_copy(cat_rows_v, cat_out.at[wid], sem_c)
    w2 = pltpu.async_copy(pooled_v, pooled_out.at[wid], sem_t)
    w1.wait()
    w2.wait()


_sc_title_cat = pl.kernel(
    _sc_title_cat_body,
    out_type=(
        jax.ShapeDtypeStruct((_NW, _BPW, _D), jnp.float32),
        jax.ShapeDtypeStruct((_NW, _BPW, _D), jnp.float32),
    ),
    mesh=plsc.VectorSubcoreMesh(core_axis_name="c", subcore_axis_name="s"),
    compiler_params=pltpu.CompilerParams(use_tc_tiling_on_sc=False),
    scratch_types=[
        pltpu.VMEM((_BPW,), jnp.int32),
        pltpu.VMEM((_S, _BPW), jnp.int32),
        pltpu.VMEM((_BPW, _D), jnp.float32),
        pltpu.VMEM((_BPW, _D), jnp.float32),
        pltpu.SemaphoreType.DMA,
        pltpu.SemaphoreType.DMA,
    ],
)


_ISTEP = 4
_NSTEP = _BPW // _ISTEP


def _sc_item_body(item_id_hbm, item_tbl_t, item_out,
                  iidx_v, buf_a, buf_b, rows_v, sem_w, sem_o):
    wid = lax.axis_index("s") * _NC + lax.axis_index("c")
    base = wid * _BPW
    iota = lax.broadcasted_iota(jnp.int32, (16,), 0)

    pltpu.sync_copy(item_id_hbm.at[pl.ds(base, _BPW)],
                    iidx_v.at[pl.ds(0, _BPW)])

    def fire(k, p, buf):
        vec = iidx_v[pl.ds(k * 2 * _ISTEP, 16)]
        for j in range(_ISTEP):
            cb = pl.multiple_of((vec[p * _ISTEP + j] // 128) * 128, 128)
            for g in range(_D // 8):
                pltpu.async_copy(
                    item_tbl_t.at[pl.ds(g * 8, 8), pl.ds(cb, 128)],
                    buf.at[pl.ds(g * 8, 8), pl.ds(j * 128, 128)], sem_w)

    def finish(k, p, buf):
        pltpu.make_async_copy(item_tbl_t.at[:, pl.ds(0, _ISTEP * 128)],
                              buf, sem_w).wait()
        vec = iidx_v[pl.ds(k * 2 * _ISTEP, 16)]
        for j in range(_ISTEP):
            row = vec[p * _ISTEP + j]
            lane = row - (row // 128) * 128
            lane_v = jnp.full((16,), lane, jnp.int32)
            slot = (2 * k + p) * _ISTEP + j
            lane_j = lane_v + j * 128
            rows_v[pl.ds(slot * _D, 16)] = plsc.load_gather(
                buf, [iota, lane_j])
            rows_v[pl.ds(slot * _D + 16, 16)] = plsc.load_gather(
                buf, [iota + 16, lane_j])

    fire(0, 0, buf_a)

    def body(k, carry):
        fire(k, 1, buf_b)
        finish(k, 0, buf_a)

        @pl.when(k < _NSTEP // 2 - 1)
        def _():
            fire(k + 1, 0, buf_a)

        finish(k, 1, buf_b)
        nw = 2 * _ISTEP * _D
        pltpu.async_copy(
            rows_v.at[pl.ds(k * nw, nw)],
            item_out.at[pl.ds(base * _D + k * nw, nw)], sem_o)
        return carry

    lax.fori_loop(0, _NSTEP // 2, body, 0)

    pltpu.make_async_copy(rows_v, item_out.at[pl.ds(0, _BPW * _D)],
                          sem_o).wait()


_sc_item = pl.kernel(
    _sc_item_body,
    out_type=jax.ShapeDtypeStruct((_B * _D,), jnp.float32),
    mesh=plsc.VectorSubcoreMesh(core_axis_name="c", subcore_axis_name="s"),
    compiler_params=pltpu.CompilerParams(use_tc_tiling_on_sc=True,
                                         needs_layout_passes=False),
    scratch_types=[
        pltpu.VMEM((_BPW + 16, ), jnp.int32),
        pltpu.VMEM((_D, _ISTEP * 128), jnp.float32),
        pltpu.VMEM((_D, _ISTEP * 128), jnp.float32),
        pltpu.VMEM((_BPW * _D,), jnp.float32),
        pltpu.SemaphoreType.DMA,
        pltpu.SemaphoreType.DMA,
    ],
)


def _dense_body(item_ref, cat_ref, pooled_ref, wi_ref, wc_ref, wp_ref,
                b_ref, out_ref):
    y = (jnp.dot(item_ref[...], wi_ref[...],
                 preferred_element_type=jnp.float32)
         + jnp.dot(cat_ref[...], wc_ref[...],
                   preferred_element_type=jnp.float32)
         + jnp.dot(pooled_ref[...], wp_ref[...],
                   preferred_element_type=jnp.float32))
    out_ref[...] = jnp.maximum(y + b_ref[...], 0.0)


_PK = 128 // _D


def kernel(item_id, category, title, item_table, category_table, title_table,
           W, b):
    cat_emb, pooled = _sc_title_cat(
        category.astype(jnp.int32),
        title.astype(jnp.int32).T,
        category_table, title_table)
    item_emb = _sc_item(item_id.astype(jnp.int32), item_table.T)

    b4 = _B // _PK
    item4 = item_emb.reshape(b4, _PK * _D)
    cat4 = cat_emb.reshape(b4, _PK * _D)
    pooled4 = pooled.reshape(b4, _PK * _D)

    eye = jnp.eye(_PK, dtype=jnp.float32)
    wi4 = jnp.kron(eye, W[0:_D, :])
    wc4 = jnp.kron(eye, W[_D:2 * _D, :])
    wp4 = jnp.kron(eye, W[2 * _D:, :] * (1.0 / _S))
    b4v = jnp.tile(b, _PK).reshape(1, _PK * 64)

    bb4 = 1024
    dense = pl.pallas_call(
        _dense_body,
        grid=(b4 // bb4,),
        in_specs=[
            pl.BlockSpec((bb4, _PK * _D), lambda i: (i, 0)),
            pl.BlockSpec((bb4, _PK * _D), lambda i: (i, 0)),
            pl.BlockSpec((bb4, _PK * _D), lambda i: (i, 0)),
            pl.BlockSpec((_PK * _D, _PK * 64), lambda i: (0, 0)),
            pl.BlockSpec((_PK * _D, _PK * 64), lambda i: (0, 0)),
            pl.BlockSpec((_PK * _D, _PK * 64), lambda i: (0, 0)),
            pl.BlockSpec((1, _PK * 64), lambda i: (0, 0)),
        ],
        out_specs=pl.BlockSpec((bb4, _PK * 64), lambda i: (i, 0)),
        out_shape=jax.ShapeDtypeStruct((b4, _PK * 64), jnp.float32),
    )
    out4 = dense(item4, cat4, pooled4, wi4, wc4, wp4, b4v)
    return out4.reshape(_B, 64)
